# baseline (speedup 1.0000x reference)
import jax
import jax.numpy as jnp
from jax.experimental import pallas as pl
from jax.experimental.pallas import tpu as pltpu
from jax.experimental.pallas import tpu_sc as plsc

_C = 256
_K = 8192
_M_BLK = 512
_W = 2048
_GW = 128


def _dist_argmax_body(x_ref, et_ref, out_ref, ebf_ref, c_ref):
    @pl.when(pl.program_id(0) == 0)
    def _():
        et = et_ref[...]
        ebf_ref[...] = (et + et).astype(jnp.bfloat16)
        c_ref[...] = jnp.sum(et * et, axis=0, keepdims=True)

    xt = x_ref[...]
    a = jnp.sum(xt * xt, axis=1, keepdims=True)
    a_b = jnp.broadcast_to(a, (_M_BLK, _W))
    xb = xt.astype(jnp.bfloat16)

    m = None
    tbest = jnp.zeros((_M_BLK, _W), dtype=jnp.float32)
    for t in range(_K // _W):
        eb = ebf_ref[:, t * _W:(t + 1) * _W]
        m2 = jax.lax.dot_general(xb, eb, (((1,), (0,)), ((), ())),
                                 preferred_element_type=jnp.float32)
        d = (m2 - a_b) - c_ref[:, t * _W:(t + 1) * _W]
        if m is None:
            m = d
        else:
            upd = d > m
            m = jnp.where(upd, d, m)
            tbest = jnp.where(upd, jnp.float32(t), tbest)

    rowmax = jnp.max(m, axis=1, keepdims=True)
    lane = jax.lax.broadcasted_iota(jnp.int32, (_M_BLK, _W), 1).astype(jnp.float32)
    gidx = tbest * jnp.float32(_W) + lane
    cand = jnp.where(m == rowmax, gidx, jnp.float32(_K))
    out_ref[...] = jnp.min(cand, axis=1, keepdims=True).astype(jnp.int32)


def _nearest_codes(x_flat, embed_t):
    m_total = x_flat.shape[0]
    return pl.pallas_call(
        _dist_argmax_body,
        grid=(m_total // _M_BLK,),
        in_specs=[
            pl.BlockSpec((_M_BLK, _C), lambda i: (i, 0)),
            pl.BlockSpec((_C, _K), lambda i: (0, 0)),
        ],
        out_specs=pl.BlockSpec((_M_BLK, 1), lambda i: (i, 0)),
        out_shape=jax.ShapeDtypeStruct((m_total, 1), jnp.int32),
        scratch_shapes=[
            pltpu.VMEM((_C, _K), jnp.bfloat16),
            pltpu.VMEM((1, _K), jnp.float32),
        ],
    )(x_flat, embed_t)


def _gather_rows(table, idx_flat):
    n = idx_flat.shape[0]
    d = table.shape[1]
    idx2 = idx_flat.reshape(1, n)
    mesh = plsc.VectorSubcoreMesh(core_axis_name="core", subcore_axis_name="subcore")

    @pl.kernel(out_type=jax.ShapeDtypeStruct((n, d), table.dtype), mesh=mesh)
    def _k(tbl_hbm, i_hbm, o_hbm):
        def body(i_vmem, o_vmem):
            pltpu.sync_copy(tbl_hbm.at[i_vmem.at[0]], o_vmem)

        pltpu.emit_pipeline(
            body,
            grid=(n // _GW,),
            in_specs=[pl.BlockSpec((1, _GW), index_map=lambda i: (0, i))],
            out_specs=[pl.BlockSpec((_GW, d), index_map=lambda i: (i, 0))],
            core_axis_name=("core", "subcore"),
            dimension_semantics=(pltpu.PARALLEL,),
        )(i_hbm, o_hbm)

    return _k(table, idx2)


def kernel(x, embed):
    b, t, c = x.shape
    x_flat = x.reshape(b * t, c)
    idx = _nearest_codes(x_flat, embed.T)[:, 0]
    quantized = _gather_rows(embed, idx).reshape(b, t, c)
    return (quantized, idx.reshape(b, t))


# speedup vs baseline: 1.0178x; 1.0178x over previous
import jax
import jax.numpy as jnp
from jax.experimental import pallas as pl
from jax.experimental.pallas import tpu as pltpu
from jax.experimental.pallas import tpu_sc as plsc

_C = 256
_K = 8192
_M_BLK = 1024
_W = 2048
_GW = 128


def _dist_argmax_body(x_ref, et_ref, out_ref, ebf_ref, c_ref):
    @pl.when(pl.program_id(0) == 0)
    def _():
        et = et_ref[...]
        ebf_ref[...] = (et + et).astype(jnp.bfloat16)
        c_ref[...] = jnp.sum(et * et, axis=0, keepdims=True)

    xt = x_ref[...]
    a = jnp.sum(xt * xt, axis=1, keepdims=True)
    a_b = jnp.broadcast_to(a, (_M_BLK, _W))
    xb = xt.astype(jnp.bfloat16)

    m = None
    tbest = jnp.zeros((_M_BLK, _W), dtype=jnp.float32)
    for t in range(_K // _W):
        eb = ebf_ref[:, t * _W:(t + 1) * _W]
        m2 = jax.lax.dot_general(xb, eb, (((1,), (0,)), ((), ())),
                                 preferred_element_type=jnp.float32)
        d = (m2 - a_b) - c_ref[:, t * _W:(t + 1) * _W]
        if m is None:
            m = d
        else:
            upd = d > m
            m = jnp.where(upd, d, m)
            tbest = jnp.where(upd, jnp.float32(t * _W), tbest)

    rowmax = jnp.max(m, axis=1, keepdims=True)
    lane = jax.lax.broadcasted_iota(jnp.int32, (_M_BLK, _W), 1).astype(jnp.float32)
    gidx = tbest + lane
    cand = jnp.where(m == rowmax, gidx, jnp.float32(_K))
    out_ref[...] = jnp.min(cand, axis=1, keepdims=True).astype(jnp.int32)


def _nearest_codes(x_flat, embed_t):
    m_total = x_flat.shape[0]
    return pl.pallas_call(
        _dist_argmax_body,
        grid=(m_total // _M_BLK,),
        in_specs=[
            pl.BlockSpec((_M_BLK, _C), lambda i: (i, 0)),
            pl.BlockSpec((_C, _K), lambda i: (0, 0)),
        ],
        out_specs=pl.BlockSpec((_M_BLK, 1), lambda i: (i, 0)),
        out_shape=jax.ShapeDtypeStruct((m_total, 1), jnp.int32),
        scratch_shapes=[
            pltpu.VMEM((_C, _K), jnp.bfloat16),
            pltpu.VMEM((1, _K), jnp.float32),
        ],
    )(x_flat, embed_t)


def _gather_rows(table, idx_flat):
    n = idx_flat.shape[0]
    d = table.shape[1]
    idx2 = idx_flat.reshape(1, n)
    mesh = plsc.VectorSubcoreMesh(core_axis_name="core", subcore_axis_name="subcore")

    @pl.kernel(out_type=jax.ShapeDtypeStruct((n, d), table.dtype), mesh=mesh)
    def _k(tbl_hbm, i_hbm, o_hbm):
        def body(i_vmem, o_vmem):
            pltpu.sync_copy(tbl_hbm.at[i_vmem.at[0]], o_vmem)

        pltpu.emit_pipeline(
            body,
            grid=(n // _GW,),
            in_specs=[pl.BlockSpec((1, _GW), index_map=lambda i: (0, i))],
            out_specs=[pl.BlockSpec((_GW, d), index_map=lambda i: (i, 0))],
            core_axis_name=("core", "subcore"),
            dimension_semantics=(pltpu.PARALLEL,),
        )(i_hbm, o_hbm)

    return _k(table, idx2)


def kernel(x, embed):
    b, t, c = x.shape
    x_flat = x.reshape(b * t, c)
    idx = _nearest_codes(x_flat, embed.T)[:, 0]
    quantized = _gather_rows(embed, idx).reshape(b, t, c)
    return (quantized, idx.reshape(b, t))


# speedup vs baseline: 1.0178x; 1.0001x over previous
import jax
import jax.numpy as jnp
from jax.experimental import pallas as pl
from jax.experimental.pallas import tpu as pltpu
from jax.experimental.pallas import tpu_sc as plsc

_C = 256
_K = 8192
_M_BLK = 1024
_W = 2048
_GW = 128


def _dist_argmax_body(x_ref, et_ref, out_ref, ebf_ref, c_ref):
    @pl.when(pl.program_id(0) == 0)
    def _():
        et = et_ref[...]
        ebf_ref[...] = (et + et).astype(jnp.bfloat16)
        c_ref[...] = jnp.sum(et * et, axis=0, keepdims=True)

    xt = x_ref[...]
    a = jnp.sum(xt * xt, axis=1, keepdims=True)
    a_b = a
    xb = xt.astype(jnp.bfloat16)

    m = None
    tbest = jnp.zeros((_M_BLK, _W), dtype=jnp.float32)
    for t in range(_K // _W):
        eb = ebf_ref[:, t * _W:(t + 1) * _W]
        m2 = jax.lax.dot_general(xb, eb, (((1,), (0,)), ((), ())),
                                 preferred_element_type=jnp.float32)
        d = (m2 - a_b) - c_ref[:, t * _W:(t + 1) * _W]
        if m is None:
            m = d
        else:
            upd = d > m
            m = jnp.where(upd, d, m)
            tbest = jnp.where(upd, jnp.float32(t * _W), tbest)

    rowmax = jnp.max(m, axis=1, keepdims=True)
    lane = jax.lax.broadcasted_iota(jnp.int32, (_M_BLK, _W), 1).astype(jnp.float32)
    gidx = tbest + lane
    cand = jnp.where(m == rowmax, gidx, jnp.float32(_K))
    out_ref[...] = jnp.min(cand, axis=1, keepdims=True).astype(jnp.int32)


def _nearest_codes(x_flat, embed_t):
    m_total = x_flat.shape[0]
    return pl.pallas_call(
        _dist_argmax_body,
        grid=(m_total // _M_BLK,),
        in_specs=[
            pl.BlockSpec((_M_BLK, _C), lambda i: (i, 0)),
            pl.BlockSpec((_C, _K), lambda i: (0, 0)),
        ],
        out_specs=pl.BlockSpec((_M_BLK, 1), lambda i: (i, 0)),
        out_shape=jax.ShapeDtypeStruct((m_total, 1), jnp.int32),
        scratch_shapes=[
            pltpu.VMEM((_C, _K), jnp.bfloat16),
            pltpu.VMEM((1, _K), jnp.float32),
        ],
    )(x_flat, embed_t)


def _gather_rows(table, idx_flat):
    n = idx_flat.shape[0]
    d = table.shape[1]
    idx2 = idx_flat.reshape(1, n)
    mesh = plsc.VectorSubcoreMesh(core_axis_name="core", subcore_axis_name="subcore")

    @pl.kernel(out_type=jax.ShapeDtypeStruct((n, d), table.dtype), mesh=mesh)
    def _k(tbl_hbm, i_hbm, o_hbm):
        def body(i_vmem, o_vmem):
            pltpu.sync_copy(tbl_hbm.at[i_vmem.at[0]], o_vmem)

        pltpu.emit_pipeline(
            body,
            grid=(n // _GW,),
            in_specs=[pl.BlockSpec((1, _GW), index_map=lambda i: (0, i))],
            out_specs=[pl.BlockSpec((_GW, d), index_map=lambda i: (i, 0))],
            core_axis_name=("core", "subcore"),
            dimension_semantics=(pltpu.PARALLEL,),
        )(i_hbm, o_hbm)

    return _k(table, idx2)


def kernel(x, embed):
    b, t, c = x.shape
    x_flat = x.reshape(b * t, c)
    idx = _nearest_codes(x_flat, embed.T)[:, 0]
    quantized = _gather_rows(embed, idx).reshape(b, t, c)
    return (quantized, idx.reshape(b, t))


# speedup vs baseline: 1.0193x; 1.0014x over previous
import jax
import jax.numpy as jnp
from jax.experimental import pallas as pl
from jax.experimental.pallas import tpu as pltpu
from jax.experimental.pallas import tpu_sc as plsc

_C = 256
_K = 8192
_M_BLK = 1024
_W = 2048
_GW = 128


def _dist_argmax_body(x_ref, et_ref, out_ref, ebf_ref, c_ref):
    @pl.when(pl.program_id(0) == 0)
    def _():
        et = et_ref[...]
        ebf_ref[...] = (et + et).astype(jnp.bfloat16)
        c_ref[...] = jnp.sum(et * et, axis=0, keepdims=True)

    xt = x_ref[...]
    a = jnp.sum(xt * xt, axis=1, keepdims=True)
    xb = xt.astype(jnp.bfloat16)

    m = None
    tbest = jnp.zeros((_M_BLK, _W), dtype=jnp.float32)
    for t in range(_K // _W):
        eb = ebf_ref[:, t * _W:(t + 1) * _W]
        m2 = jax.lax.dot_general(xb, eb, (((1,), (0,)), ((), ())),
                                 preferred_element_type=jnp.float32)
        d = (m2 - a) - c_ref[:, t * _W:(t + 1) * _W]
        if m is None:
            m = d
        else:
            upd = d > m
            m = jnp.where(upd, d, m)
            tbest = jnp.where(upd, jnp.float32(t * _W), tbest)

    rowmax = jnp.max(m, axis=1, keepdims=True)
    lane = jax.lax.broadcasted_iota(jnp.int32, (_M_BLK, _W), 1).astype(jnp.float32)
    gidx = tbest + lane
    cand = jnp.where(m == rowmax, gidx, jnp.float32(_K))
    out_ref[...] = jnp.min(cand, axis=1, keepdims=True).astype(jnp.int32)


def _nearest_codes(x_flat, embed_t):
    m_total = x_flat.shape[0]
    return pl.pallas_call(
        _dist_argmax_body,
        grid=(m_total // _M_BLK,),
        in_specs=[
            pl.BlockSpec((_M_BLK, _C), lambda i: (i, 0)),
            pl.BlockSpec((_C, _K), lambda i: (0, 0)),
        ],
        out_specs=pl.BlockSpec((_M_BLK, 1), lambda i: (i, 0)),
        out_shape=jax.ShapeDtypeStruct((m_total, 1), jnp.int32),
        scratch_shapes=[
            pltpu.VMEM((_C, _K), jnp.bfloat16),
            pltpu.VMEM((1, _K), jnp.float32),
        ],
    )(x_flat, embed_t)


def _gather_rows(table, idx_flat):
    n = idx_flat.shape[0]
    d = table.shape[1]
    idx2 = idx_flat.reshape(1, n)
    mesh = plsc.VectorSubcoreMesh(core_axis_name="core", subcore_axis_name="subcore")

    @pl.kernel(out_type=jax.ShapeDtypeStruct((n, d), table.dtype), mesh=mesh)
    def _k(tbl_hbm, i_hbm, o_hbm):
        def body(i_vmem, o_vmem):
            pltpu.sync_copy(tbl_hbm.at[i_vmem.at[0]], o_vmem)

        pltpu.emit_pipeline(
            body,
            grid=(n // _GW,),
            in_specs=[pl.BlockSpec((1, _GW), index_map=lambda i: (0, i))],
            out_specs=[pl.BlockSpec((_GW, d), index_map=lambda i: (i, 0))],
            core_axis_name=("core", "subcore"),
            dimension_semantics=(pltpu.PARALLEL,),
        )(i_hbm, o_hbm)

    return _k(table, idx2)


def kernel(x, embed):
    b, t, c = x.shape
    x_flat = x.reshape(b * t, c)
    idx = _nearest_codes(x_flat, embed.T)[:, 0]
    quantized = _gather_rows(embed, idx).reshape(b, t, c)
    return (quantized, idx.reshape(b, t))
